# baseline (speedup 1.0000x reference)
import functools

import jax
import jax.numpy as jnp
from jax import lax
from jax.experimental import pallas as pl
from jax.experimental.pallas import tpu as pltpu
from jax.experimental.pallas import tpu_sc as plsc

NUM_CORES = 2
NUM_SUBCORES = 16
NUM_WORKERS = NUM_CORES * NUM_SUBCORES
CHUNK = 128
EMBED_DIM = 128
ROW_PAD = 136


def _fuse_table_body(emb_ref, fr_ref, pr_ref, w1_ref, w2_ref, b_ref, out_ref):
    out_ref[...] = (
        emb_ref[...]
        + jnp.dot(fr_ref[...], w1_ref[...], preferred_element_type=jnp.float32)
        + jnp.dot(pr_ref[...], w2_ref[...], preferred_element_type=jnp.float32)
        + b_ref[...]
    )


def _fuse_table(emb, fr, pr, w1, w2, b):
    return pl.pallas_call(
        _fuse_table_body,
        out_shape=jax.ShapeDtypeStruct((ROW_PAD, EMBED_DIM), jnp.float32),
    )(emb, fr, pr, w1, w2, b)


@functools.partial(jax.jit, static_argnums=(2, 3))
def _gather_rows(table, idx2d, n_chunks_total, n_chunks_per_worker):
    mesh = plsc.VectorSubcoreMesh(core_axis_name="c", subcore_axis_name="s")

    nbuf = 2
    assert n_chunks_per_worker % nbuf == 0 and n_chunks_per_worker > nbuf

    @functools.partial(
        pl.kernel,
        mesh=mesh,
        out_type=jax.ShapeDtypeStruct((n_chunks_total * CHUNK, EMBED_DIM),
                                      jnp.float32),
        scratch_types=[
            pltpu.VMEM((ROW_PAD, EMBED_DIM), jnp.float32),
            pltpu.VMEM((n_chunks_per_worker, CHUNK), jnp.int32),
            [pltpu.VMEM((CHUNK, EMBED_DIM), jnp.float32)] * nbuf,
            [pltpu.SemaphoreType.DMA] * nbuf,
        ],
    )
    def gather(table_hbm, idx_hbm, out_hbm, table_v, idx_v, rows, ssem):
        wid = lax.axis_index("s") * NUM_CORES + lax.axis_index("c")
        chunk0 = wid * n_chunks_per_worker
        row0 = chunk0 * CHUNK

        def out_slice(g):
            return out_hbm.at[pl.ds(row0 + g * CHUNK, CHUNK)]

        pltpu.sync_copy(table_hbm, table_v)
        pltpu.sync_copy(idx_hbm.at[pl.ds(chunk0, n_chunks_per_worker)], idx_v)

        @pl.loop(0, n_chunks_per_worker, step=nbuf)
        def outer(g0):
            for bi in range(nbuf):
                g = g0 + bi

                @pl.when(g0 > 0)
                def _():
                    pltpu.make_async_copy(rows[bi], out_slice(g),
                                          ssem[bi]).wait()

                @pl.loop(0, CHUNK // 16)
                def fill(j16):
                    iv = idx_v[g, pl.ds(j16 * 16, 16)]
                    for t in range(16):
                        i = iv[t]
                        for k in range(EMBED_DIM // 16):
                            sl = pl.ds(k * 16, 16)
                            rows[bi][j16 * 16 + t, sl] = table_v[i, sl]

                pltpu.async_copy(rows[bi], out_slice(g), ssem[bi])

        for bi in range(nbuf):
            g_last = n_chunks_per_worker - nbuf + bi
            pltpu.make_async_copy(rows[bi], out_slice(g_last),
                                  ssem[bi]).wait()

    return gather(table, idx2d)


def kernel(instrument_indices, embedding_table, frequency_ranges,
           instrument_properties, W, b):
    batch, seq = instrument_indices.shape
    pad = ROW_PAD - embedding_table.shape[0]
    emb = jnp.pad(embedding_table, ((0, pad), (0, 0)))
    fr = jnp.pad(frequency_ranges, ((0, pad), (0, 0)))
    pr = jnp.pad(instrument_properties, ((0, pad), (0, 0)))
    fused = _fuse_table(emb, fr, pr, W[:fr.shape[1]], W[fr.shape[1]:],
                        b.reshape(1, EMBED_DIM))

    total = batch * seq
    n_chunks_total = total // CHUNK
    n_chunks_per_worker = n_chunks_total // NUM_WORKERS
    idx2d = instrument_indices.reshape(n_chunks_total, CHUNK).astype(jnp.int32)
    out = _gather_rows(fused, idx2d, n_chunks_total, n_chunks_per_worker)
    return out.reshape(batch, seq, EMBED_DIM)


# speedup vs baseline: 1.8762x; 1.8762x over previous
import functools

import jax
import jax.numpy as jnp
from jax import lax
from jax.experimental import pallas as pl
from jax.experimental.pallas import tpu as pltpu
from jax.experimental.pallas import tpu_sc as plsc

NUM_CORES = 2
NUM_SUBCORES = 16
NUM_WORKERS = NUM_CORES * NUM_SUBCORES
CHUNK = 128
EMBED_DIM = 128
ROW_PAD = 136


def _fuse_table_body(emb_ref, fr_ref, pr_ref, w1_ref, w2_ref, b_ref, out_ref):
    out_ref[...] = (
        emb_ref[...]
        + jnp.dot(fr_ref[...], w1_ref[...], preferred_element_type=jnp.float32)
        + jnp.dot(pr_ref[...], w2_ref[...], preferred_element_type=jnp.float32)
        + b_ref[...]
    )


def _fuse_table(emb, fr, pr, w1, w2, b):
    return pl.pallas_call(
        _fuse_table_body,
        out_shape=jax.ShapeDtypeStruct((ROW_PAD, EMBED_DIM), jnp.float32),
    )(emb, fr, pr, w1, w2, b)


@functools.partial(jax.jit, static_argnums=(2, 3))
def _gather_rows(table, idx2d, n_chunks_total, n_chunks_per_worker):
    mesh = plsc.VectorSubcoreMesh(core_axis_name="c", subcore_axis_name="s")

    n_pairs = n_chunks_per_worker // 2
    assert n_pairs % 2 == 0 and n_pairs > 4

    @functools.partial(
        pl.kernel,
        mesh=mesh,
        out_type=jax.ShapeDtypeStruct((n_chunks_total * CHUNK, EMBED_DIM),
                                      jnp.float32),
        scratch_types=[
            pltpu.VMEM((ROW_PAD, EMBED_DIM), jnp.float32),
            pltpu.VMEM((n_chunks_per_worker, CHUNK), jnp.int32),
            [pltpu.VMEM((CHUNK, EMBED_DIM), jnp.float32)] * 2,
            [pltpu.VMEM((CHUNK, EMBED_DIM), jnp.float32)] * 2,
            [pltpu.SemaphoreType.DMA] * 2,
            [pltpu.SemaphoreType.DMA] * 2,
            [pltpu.SemaphoreType.DMA] * 2,
        ],
    )
    def gather(table_hbm, idx_hbm, out_hbm, table_v, idx_v, sbuf, cbuf,
               gsem, ssemS, ssemC):
        wid = lax.axis_index("s") * NUM_CORES + lax.axis_index("c")
        chunk0 = wid * n_chunks_per_worker
        row0 = chunk0 * CHUNK
        base = wid * ROW_PAD + jnp.zeros((16,), jnp.int32)

        def out_slice(g):
            return out_hbm.at[pl.ds(row0 + g * CHUNK, CHUNK)]

        def gather_start(p, bi):
            pltpu.async_copy(table_hbm.at[idx_v.at[2 * p]], sbuf[bi],
                             gsem[bi])

        def gather_wait(p, bi):
            pltpu.make_async_copy(table_hbm.at[idx_v.at[2 * p]], sbuf[bi],
                                  gsem[bi]).wait()

        pltpu.sync_copy(table_hbm.at[pl.ds(wid * ROW_PAD, ROW_PAD)], table_v)
        pltpu.sync_copy(idx_hbm.at[pl.ds(chunk0, n_chunks_per_worker)], idx_v)

        @pl.loop(0, n_chunks_per_worker)
        def adjust(g):
            for j in range(CHUNK // 16):
                sl = pl.ds(j * 16, 16)
                idx_v[g, sl] = idx_v[g, sl] + base

        for bi in range(2):
            gather_start(bi, bi)

        @pl.loop(0, n_pairs, step=2)
        def pair(p0):
            for bi in range(2):
                p = p0 + bi
                gs = 2 * p
                gc = 2 * p + 1
                gather_wait(p, bi)
                pltpu.async_copy(sbuf[bi], out_slice(gs), ssemS[bi])

                @pl.when(p >= 2)
                def _():
                    pltpu.make_async_copy(cbuf[bi], out_slice(gc),
                                          ssemC[bi]).wait()

                @pl.loop(0, CHUNK // 16)
                def fill(j16):
                    iv = idx_v[gc, pl.ds(j16 * 16, 16)] - base
                    for t in range(16):
                        i = iv[t]
                        for k in range(EMBED_DIM // 16):
                            sl = pl.ds(k * 16, 16)
                            cbuf[bi][j16 * 16 + t, sl] = table_v[i, sl]

                pltpu.async_copy(cbuf[bi], out_slice(gc), ssemC[bi])
                pltpu.make_async_copy(sbuf[bi], out_slice(gs),
                                      ssemS[bi]).wait()

                @pl.when(p + 2 < n_pairs)
                def _():
                    gather_start(p + 2, bi)

        for bi in range(2):
            p_last = n_pairs - 2 + bi
            pltpu.make_async_copy(cbuf[bi], out_slice(2 * p_last + 1),
                                  ssemC[bi]).wait()

    return gather(table, idx2d)


def kernel(instrument_indices, embedding_table, frequency_ranges,
           instrument_properties, W, b):
    batch, seq = instrument_indices.shape
    pad = ROW_PAD - embedding_table.shape[0]
    emb = jnp.pad(embedding_table, ((0, pad), (0, 0)))
    fr = jnp.pad(frequency_ranges, ((0, pad), (0, 0)))
    pr = jnp.pad(instrument_properties, ((0, pad), (0, 0)))
    fused = _fuse_table(emb, fr, pr, W[:fr.shape[1]], W[fr.shape[1]:],
                        b.reshape(1, EMBED_DIM))
    fused = jnp.tile(fused, (NUM_WORKERS, 1))

    total = batch * seq
    n_chunks_total = total // CHUNK
    n_chunks_per_worker = n_chunks_total // NUM_WORKERS
    idx2d = instrument_indices.reshape(n_chunks_total, CHUNK).astype(jnp.int32)
    out = _gather_rows(fused, idx2d, n_chunks_total, n_chunks_per_worker)
    return out.reshape(batch, seq, EMBED_DIM)


# speedup vs baseline: 2.0136x; 1.0733x over previous
import functools

import jax
import jax.numpy as jnp
from jax import lax
from jax.experimental import pallas as pl
from jax.experimental.pallas import tpu as pltpu
from jax.experimental.pallas import tpu_sc as plsc

NUM_CORES = 2
NUM_SUBCORES = 16
NUM_WORKERS = NUM_CORES * NUM_SUBCORES
CHUNK = 128
EMBED_DIM = 128
ROW_PAD = 136


def _fuse_table_body(emb_ref, fr_ref, pr_ref, w1_ref, w2_ref, b_ref, out_ref):
    out_ref[...] = (
        emb_ref[...]
        + jnp.dot(fr_ref[...], w1_ref[...], preferred_element_type=jnp.float32)
        + jnp.dot(pr_ref[...], w2_ref[...], preferred_element_type=jnp.float32)
        + b_ref[...]
    )


def _fuse_table(emb, fr, pr, w1, w2, b):
    return pl.pallas_call(
        _fuse_table_body,
        out_shape=jax.ShapeDtypeStruct((ROW_PAD, EMBED_DIM), jnp.float32),
    )(emb, fr, pr, w1, w2, b)


@functools.partial(jax.jit, static_argnums=(2, 3))
def _gather_rows(table, idx2d, n_chunks_total, n_chunks_per_worker):
    mesh = plsc.VectorSubcoreMesh(core_axis_name="c", subcore_axis_name="s")

    nbuf = 5
    assert n_chunks_per_worker % nbuf == 0 and n_chunks_per_worker > nbuf

    @functools.partial(
        pl.kernel,
        mesh=mesh,
        out_type=jax.ShapeDtypeStruct((n_chunks_total * CHUNK, EMBED_DIM),
                                      jnp.float32),
        scratch_types=[
            pltpu.VMEM((n_chunks_per_worker, CHUNK), jnp.int32),
            [pltpu.VMEM((CHUNK, EMBED_DIM), jnp.float32)] * nbuf,
            [pltpu.SemaphoreType.DMA] * nbuf,
            [pltpu.SemaphoreType.DMA] * nbuf,
        ],
    )
    def gather(table_hbm, idx_hbm, out_hbm, idx_v, rows, gsem, ssem):
        wid = lax.axis_index("s") * NUM_CORES + lax.axis_index("c")
        row0 = wid * n_chunks_per_worker * CHUNK

        def gather_start(g, bi):
            pltpu.async_copy(table_hbm.at[idx_v.at[g]], rows[bi], gsem[bi])

        def gather_wait(g, bi):
            pltpu.make_async_copy(table_hbm.at[idx_v.at[g]], rows[bi],
                                  gsem[bi]).wait()

        def out_slice(g):
            return out_hbm.at[pl.ds(row0 + g * CHUNK, CHUNK)]

        pltpu.sync_copy(idx_hbm.at[wid], idx_v)

        base = wid * ROW_PAD + jnp.zeros((16,), jnp.int32)

        @pl.loop(0, n_chunks_per_worker)
        def adjust(g):
            for j in range(CHUNK // 16):
                sl = pl.ds(j * 16, 16)
                idx_v[g, sl] = idx_v[g, sl] + base

        for bi in range(nbuf):
            gather_start(bi, bi)

        @pl.loop(0, n_chunks_per_worker, step=nbuf)
        def outer(g0):
            for bi in range(nbuf):
                g = g0 + bi
                gather_wait(g, bi)
                pltpu.async_copy(rows[bi], out_slice(g), ssem[bi])

                @pl.when(g + nbuf < n_chunks_per_worker)
                def _():
                    pltpu.make_async_copy(rows[bi], out_slice(g),
                                          ssem[bi]).wait()
                    gather_start(g + nbuf, bi)

        for bi in range(nbuf):
            g_last = n_chunks_per_worker - nbuf + bi
            pltpu.make_async_copy(rows[bi], out_slice(g_last),
                                  ssem[bi]).wait()

    return gather(table, idx2d)


def kernel(instrument_indices, embedding_table, frequency_ranges,
           instrument_properties, W, b):
    batch, seq = instrument_indices.shape
    pad = ROW_PAD - embedding_table.shape[0]
    emb = jnp.pad(embedding_table, ((0, pad), (0, 0)))
    fr = jnp.pad(frequency_ranges, ((0, pad), (0, 0)))
    pr = jnp.pad(instrument_properties, ((0, pad), (0, 0)))
    fused = _fuse_table(emb, fr, pr, W[:fr.shape[1]], W[fr.shape[1]:],
                        b.reshape(1, EMBED_DIM))
    fused = jnp.tile(fused, (NUM_WORKERS, 1))

    total = batch * seq
    n_chunks_total = total // CHUNK
    n_chunks_per_worker = n_chunks_total // NUM_WORKERS
    idx2d = instrument_indices.reshape(
        NUM_WORKERS, n_chunks_per_worker, CHUNK).astype(jnp.int32)
    out = _gather_rows(fused, idx2d, n_chunks_total, n_chunks_per_worker)
    return out.reshape(batch, seq, EMBED_DIM)
